# baseline (speedup 1.0000x reference)
import functools
import math

import jax
import jax.numpy as jnp
from jax import lax
from jax.experimental import pallas as pl
from jax.experimental.pallas import tpu as pltpu
from jax.experimental.pallas import tpu_sc as plsc

D_MODEL = 64
SCALE = math.sqrt(D_MODEL)
NBUF = 2


@functools.lru_cache(maxsize=None)
def _build_lookup(b: int, s: int, d: int):
    info = plsc.get_sparse_core_info()
    nc, ns = info.num_cores, info.num_subcores
    nw = nc * ns
    assert b % nw == 0 and s % 8 == 0
    b_per_w = b // nw
    n_per_w = b_per_w * s

    mesh = plsc.VectorSubcoreMesh(core_axis_name="c", subcore_axis_name="s")

    @functools.partial(
        pl.kernel,
        mesh=mesh,
        out_type=jax.ShapeDtypeStruct((b, s, d), jnp.float32),
        scratch_types=[
            pltpu.VMEM((n_per_w,), jnp.int32),
            pltpu.VMEM((NBUF, s, d), jnp.float32),
            pltpu.VMEM((NBUF, s, d), jnp.float32),
            pltpu.SemaphoreType.DMA,
            pltpu.SemaphoreType.DMA,
        ],
        compiler_params=pltpu.CompilerParams(use_tc_tiling_on_sc=False),
    )
    def lookup(idx_hbm, table_hbm, out_hbm, idx_v, buf, obuf, sem_g, sem_o):
        wid = lax.axis_index("s") * nc + lax.axis_index("c")
        b0 = wid * b_per_w
        pltpu.sync_copy(idx_hbm.at[pl.ds(wid * n_per_w, n_per_w)], idx_v)

        def gather(c, slot):
            return pltpu.make_async_copy(
                table_hbm.at[idx_v.at[pl.ds(c * s, s)]], buf.at[slot], sem_g)

        def put(c, slot):
            return pltpu.make_async_copy(
                obuf.at[slot], out_hbm.at[b0 + c], sem_o)

        for c in range(NBUF):
            gather(c, c).start()

        def chunk_body(c, carry):
            slot = lax.rem(c, NBUF)
            gather(c, slot).wait()

            @pl.when(c >= NBUF)
            def _():
                put(c - NBUF, slot).wait()

            def row_body(j4, c2):
                j = j4 * 4
                for dj in range(4):
                    for k in range(d // 16):
                        sl = pl.ds(k * 16, 16)
                        obuf[slot, j + dj, sl] = buf[slot, j + dj, sl] * SCALE
                return c2

            lax.fori_loop(0, s // 4, row_body, 0)

            @pl.when(c + NBUF < b_per_w)
            def _():
                gather(c + NBUF, slot).start()

            put(c, slot).start()
            return carry

        lax.fori_loop(0, b_per_w, chunk_body, 0)

        for c in range(b_per_w - NBUF, b_per_w):
            put(c, c % NBUF).wait()

    return lookup


def kernel(x, table):
    b, s = x.shape
    d = table.shape[1]
    idx = x.reshape(b * s).astype(jnp.int32)
    return _build_lookup(b, s, d)(idx, table)
